# Initial kernel scaffold; baseline (speedup 1.0000x reference)
import jax
import jax.numpy as jnp
from jax.experimental import pallas as pl


def kernel(sequence, table):
    raise NotImplementedError("write your pallas kernel here")


# speedup vs baseline: 3.0810x; 3.0810x over previous
import functools

import jax
import jax.numpy as jnp
from jax import lax
from jax.experimental import pallas as pl
from jax.experimental.pallas import tpu as pltpu
from jax.experimental.pallas import tpu_sc as plsc


def kernel(sequence, table):
    batch, seq_len = sequence.shape
    _, hidden = table.shape

    info = plsc.get_sparse_core_info()
    num_workers = info.num_cores * info.num_subcores
    rows_per_worker = seq_len // num_workers
    chunk = min(64, rows_per_worker)
    n_chunks = rows_per_worker // chunk

    mesh = plsc.VectorSubcoreMesh(core_axis_name="c", subcore_axis_name="s")

    @functools.partial(
        pl.kernel,
        mesh=mesh,
        out_type=jax.ShapeDtypeStruct((batch, seq_len, hidden), jnp.float32),
        scratch_types=[pltpu.VMEM((chunk, hidden), jnp.float32)],
    )
    def body(table_hbm, out_hbm, buf):
        wid = lax.axis_index("s") * info.num_cores + lax.axis_index("c")
        for i in range(n_chunks):
            base = (wid * n_chunks + i) * chunk
            pltpu.sync_copy(table_hbm.at[pl.ds(base, chunk)], buf)
            for b in range(batch):
                pltpu.sync_copy(buf, out_hbm.at[b, pl.ds(base, chunk)])

    return body(table)
